# baseline (speedup 1.0000x reference)
import functools

import jax
import jax.numpy as jnp
from jax import lax
from jax.experimental import pallas as pl
from jax.experimental.pallas import tpu as pltpu
from jax.experimental.pallas import tpu_sc as plsc

NUM_STOCKS = 100000
CELL_SIZE = 64
BATCH = 16384

NC, NS = 2, 16
NW = NC * NS
B_PER_W = BATCH // NW
W_PER_TILE = B_PER_W * CELL_SIZE


def _encoder_gather(idx_flat, e0_flat, e1_flat):
    mesh = plsc.VectorSubcoreMesh(core_axis_name="c", subcore_axis_name="s")
    out_t = (
        jax.ShapeDtypeStruct((BATCH * CELL_SIZE,), jnp.float32),
        jax.ShapeDtypeStruct((BATCH * CELL_SIZE,), jnp.float32),
    )

    @functools.partial(
        pl.kernel,
        out_type=out_t,
        mesh=mesh,
        scratch_types=[
            pltpu.VMEM((B_PER_W,), jnp.int32),
            pltpu.VMEM((W_PER_TILE,), jnp.float32),
            pltpu.VMEM((W_PER_TILE,), jnp.float32),
            pltpu.SemaphoreType.DMA,
            pltpu.SemaphoreType.DMA,
            pltpu.SemaphoreType.DMA,
            pltpu.SemaphoreType.DMA,
        ],
    )
    def k(e0_hbm, e1_hbm, idx_hbm, o0_hbm, o1_hbm,
          idx_v, rows0_v, rows1_v, sem_g0, sem_g1, sem_w0, sem_w1):
        wid = lax.axis_index("s") * NC + lax.axis_index("c")
        base = wid * B_PER_W
        pltpu.sync_copy(idx_hbm.at[pl.ds(base, B_PER_W)], idx_v)

        @pl.loop(0, B_PER_W, step=16)
        def _(j):
            v = idx_v[pl.ds(j, 16)]
            for t in range(16):
                src = v[t] * CELL_SIZE
                dst = (j + t) * CELL_SIZE
                pltpu.make_async_copy(
                    e0_hbm.at[pl.ds(src, CELL_SIZE)],
                    rows0_v.at[pl.ds(dst, CELL_SIZE)],
                    sem_g0).start()
                pltpu.make_async_copy(
                    e1_hbm.at[pl.ds(src, CELL_SIZE)],
                    rows1_v.at[pl.ds(dst, CELL_SIZE)],
                    sem_g1).start()

        obase = base * CELL_SIZE
        pltpu.make_async_copy(
            o0_hbm.at[pl.ds(obase, W_PER_TILE)], rows0_v, sem_g0).wait()
        w0 = pltpu.async_copy(
            rows0_v, o0_hbm.at[pl.ds(obase, W_PER_TILE)], sem_w0)
        pltpu.make_async_copy(
            o1_hbm.at[pl.ds(obase, W_PER_TILE)], rows1_v, sem_g1).wait()
        w1 = pltpu.async_copy(
            rows1_v, o1_hbm.at[pl.ds(obase, W_PER_TILE)], sem_w1)
        w0.wait()
        w1.wait()

    return k(e0_flat, e1_flat, idx_flat)


def kernel(Stock_ID, emb0, emb1):
    idx_flat = Stock_ID.reshape(BATCH).astype(jnp.int32)
    o0, o1 = _encoder_gather(idx_flat, emb0.reshape(-1), emb1.reshape(-1))
    return (o0.reshape(BATCH, CELL_SIZE), o1.reshape(BATCH, CELL_SIZE))
